# Initial kernel scaffold; baseline (speedup 1.0000x reference)
import jax
import jax.numpy as jnp
from jax.experimental import pallas as pl


def kernel(x, pool_inds):
    raise NotImplementedError("write your pallas kernel here")


# speedup vs baseline: 2.8649x; 2.8649x over previous
import functools

import jax
import jax.numpy as jnp
from jax import lax
from jax.experimental import pallas as pl
from jax.experimental.pallas import tpu as pltpu
from jax.experimental.pallas import tpu_sc as plsc

N_ROWS_IN = 50000
D = 128
N_OUT = 25000
GROUP = 16

LANES = 16
ROWS_PER_BLOCK = 8
IDX_PER_BLOCK = ROWS_PER_BLOCK * GROUP
N_BLOCKS = N_OUT // ROWS_PER_BLOCK
N_WORKERS = 32
BLOCKS_PER_WORKER = -(-N_BLOCKS // N_WORKERS)


def _sc_max_pool(x_padded, idx_flat):
    mesh = plsc.VectorSubcoreMesh(core_axis_name="c", subcore_axis_name="s")

    @functools.partial(
        pl.kernel,
        out_type=jax.ShapeDtypeStruct((N_OUT, D), jnp.float32),
        mesh=mesh,
        scratch_types=[
            pltpu.VMEM((IDX_PER_BLOCK,), jnp.int32),
            pltpu.VMEM((IDX_PER_BLOCK, D), jnp.float32),
            pltpu.VMEM((ROWS_PER_BLOCK, D), jnp.float32),
            pltpu.SemaphoreType.DMA,
        ],
    )
    def kern(x_hbm, idx_hbm, out_hbm, idx_v, rows_v, out_v, sem):
        wid = lax.axis_index("s") * 2 + lax.axis_index("c")

        @pl.loop(0, BLOCKS_PER_WORKER)
        def _(i):
            b = wid + i * N_WORKERS

            @pl.when(b < N_BLOCKS)
            def _():
                pltpu.sync_copy(idx_hbm.at[pl.ds(b * IDX_PER_BLOCK, IDX_PER_BLOCK)], idx_v)
                pltpu.async_copy(x_hbm.at[idx_v], rows_v, sem).wait()

                @pl.loop(0, ROWS_PER_BLOCK)
                def _(r):
                    base = r * GROUP
                    for c in range(D // LANES):
                        sl = pl.ds(c * LANES, LANES)
                        acc = rows_v[base, sl]
                        for j in range(1, GROUP):
                            acc = jnp.maximum(acc, rows_v[base + j, sl])
                        out_v[r, sl] = acc

                pltpu.sync_copy(out_v, out_hbm.at[pl.ds(b * ROWS_PER_BLOCK, ROWS_PER_BLOCK)])

    return kern(x_padded, idx_flat)


def kernel(x, pool_inds):
    x_padded = jnp.concatenate([x, jnp.zeros((1, D), dtype=x.dtype)], axis=0)
    idx_flat = pool_inds.astype(jnp.int32).reshape(N_OUT * GROUP)
    return _sc_max_pool(x_padded, idx_flat)


# speedup vs baseline: 3.4866x; 1.2170x over previous
import functools

import jax
import jax.numpy as jnp
from jax import lax
from jax.experimental import pallas as pl
from jax.experimental.pallas import tpu as pltpu
from jax.experimental.pallas import tpu_sc as plsc

N_ROWS_IN = 50000
D = 128
N_OUT = 25000
GROUP = 16

LANES = 16
ROWS_PER_BLOCK = 8
IDX_PER_BLOCK = ROWS_PER_BLOCK * GROUP
N_WORKERS = 32
BLOCKS_PER_WORKER = -(-N_OUT // (ROWS_PER_BLOCK * N_WORKERS))
N_BLOCKS_PAD = BLOCKS_PER_WORKER * N_WORKERS
N_OUT_PAD = N_BLOCKS_PAD * ROWS_PER_BLOCK
NBUF = 2


def _tree_max(vals):
    while len(vals) > 1:
        vals = [jnp.maximum(a, b) for a, b in zip(vals[::2], vals[1::2])]
    return vals[0]


def _sc_max_pool(x_padded, idx_flat):
    mesh = plsc.VectorSubcoreMesh(core_axis_name="c", subcore_axis_name="s")

    @functools.partial(
        pl.kernel,
        out_type=jax.ShapeDtypeStruct((N_OUT_PAD, D), jnp.float32),
        mesh=mesh,
        scratch_types=[
            pltpu.VMEM((BLOCKS_PER_WORKER * IDX_PER_BLOCK,), jnp.int32),
            pltpu.VMEM((NBUF, IDX_PER_BLOCK, D), jnp.float32),
            pltpu.VMEM((NBUF, ROWS_PER_BLOCK, D), jnp.float32),
            pltpu.SemaphoreType.DMA,
            pltpu.SemaphoreType.DMA,
            pltpu.SemaphoreType.DMA,
            pltpu.SemaphoreType.DMA,
        ],
    )
    def kern(x_hbm, idx_hbm, out_hbm, idx_all, rows, outv, g0, g1, o0, o1):
        gsems = [g0, g1]
        osems = [o0, o1]
        wid = lax.axis_index("s") * 2 + lax.axis_index("c")
        base_blk = wid * BLOCKS_PER_WORKER

        pltpu.sync_copy(
            idx_hbm.at[pl.ds(base_blk * IDX_PER_BLOCK,
                             BLOCKS_PER_WORKER * IDX_PER_BLOCK)],
            idx_all,
        )

        def gather_start(slot, i):
            idx_slice = idx_all.at[pl.ds(i * IDX_PER_BLOCK, IDX_PER_BLOCK)]
            pltpu.async_copy(x_hbm.at[idx_slice], rows.at[slot], gsems[slot])

        def gather_wait(slot):
            pltpu.make_async_copy(
                x_hbm.at[pl.ds(0, IDX_PER_BLOCK)], rows.at[slot], gsems[slot]
            ).wait()

        def out_wait(slot):
            pltpu.make_async_copy(
                outv.at[slot], out_hbm.at[pl.ds(0, ROWS_PER_BLOCK)], osems[slot]
            ).wait()

        gather_start(0, 0)
        gather_start(1, 1)

        @pl.loop(0, BLOCKS_PER_WORKER // NBUF)
        def _(p):
            for slot in range(NBUF):
                i = p * NBUF + slot
                gather_wait(slot)

                @pl.when(i >= NBUF)
                def _():
                    out_wait(slot)

                rbuf = rows.at[slot]
                obuf = outv.at[slot]

                @pl.loop(0, ROWS_PER_BLOCK)
                def _(r):
                    base = r * GROUP
                    for c in range(D // LANES):
                        sl = pl.ds(c * LANES, LANES)
                        obuf[r, sl] = _tree_max(
                            [rbuf[base + j, sl] for j in range(GROUP)]
                        )

                pltpu.async_copy(
                    obuf,
                    out_hbm.at[pl.ds((base_blk + i) * ROWS_PER_BLOCK,
                                     ROWS_PER_BLOCK)],
                    osems[slot],
                )

                @pl.when(i + NBUF < BLOCKS_PER_WORKER)
                def _():
                    gather_start(slot, i + NBUF)

        out_wait(0)
        out_wait(1)

    return kern(x_padded, idx_flat)


def kernel(x, pool_inds):
    x_padded = jnp.concatenate([x, jnp.zeros((1, D), dtype=x.dtype)], axis=0)
    idx_flat = pool_inds.astype(jnp.int32).reshape(N_OUT * GROUP)
    idx_flat = jnp.concatenate(
        [idx_flat, jnp.zeros(N_BLOCKS_PAD * IDX_PER_BLOCK - N_OUT * GROUP,
                             dtype=jnp.int32)]
    )
    return _sc_max_pool(x_padded, idx_flat)[:N_OUT]


# speedup vs baseline: 3.6160x; 1.0371x over previous
import functools

import jax
import jax.numpy as jnp
from jax import lax
from jax.experimental import pallas as pl
from jax.experimental.pallas import tpu as pltpu
from jax.experimental.pallas import tpu_sc as plsc

N_ROWS_IN = 50000
D = 128
N_OUT = 25000
GROUP = 16

LANES = 16
ROWS_PER_BLOCK = 8
IDX_PER_BLOCK = ROWS_PER_BLOCK * GROUP
N_WORKERS = 32
BLOCKS_PER_WORKER = -(-N_OUT // (ROWS_PER_BLOCK * N_WORKERS))
N_BLOCKS_PAD = BLOCKS_PER_WORKER * N_WORKERS
N_OUT_PAD = N_BLOCKS_PAD * ROWS_PER_BLOCK
NBUF = 2


def _tree_max(vals):
    while len(vals) > 1:
        vals = [jnp.maximum(a, b) for a, b in zip(vals[::2], vals[1::2])]
    return vals[0]


def _sc_max_pool(x_padded, idx_flat):
    mesh = plsc.VectorSubcoreMesh(core_axis_name="c", subcore_axis_name="s")

    @functools.partial(
        pl.kernel,
        out_type=jax.ShapeDtypeStruct((N_OUT_PAD, D), jnp.float32),
        mesh=mesh,
        scratch_types=[
            pltpu.VMEM((BLOCKS_PER_WORKER * IDX_PER_BLOCK,), jnp.int32),
            pltpu.VMEM((NBUF, IDX_PER_BLOCK, D), jnp.float32),
            pltpu.VMEM((NBUF, ROWS_PER_BLOCK, D), jnp.float32),
            pltpu.SemaphoreType.DMA,
            pltpu.SemaphoreType.DMA,
            pltpu.SemaphoreType.DMA,
            pltpu.SemaphoreType.DMA,
        ],
    )
    def kern(x_hbm, idx_hbm, out_hbm, idx_all, rows, outv, g0, g1, o0, o1):
        gsems = [g0, g1]
        osems = [o0, o1]
        wid = lax.axis_index("s") * 2 + lax.axis_index("c")
        base_blk = wid * BLOCKS_PER_WORKER

        pltpu.sync_copy(
            idx_hbm.at[pl.ds(base_blk * IDX_PER_BLOCK,
                             BLOCKS_PER_WORKER * IDX_PER_BLOCK)],
            idx_all,
        )

        def gather_start(slot, i):
            idx_slice = idx_all.at[pl.ds(i * IDX_PER_BLOCK, IDX_PER_BLOCK)]
            pltpu.async_copy(x_hbm.at[idx_slice], rows.at[slot], gsems[slot])

        def gather_wait(slot):
            pltpu.make_async_copy(
                x_hbm.at[pl.ds(0, IDX_PER_BLOCK)], rows.at[slot], gsems[slot]
            ).wait()

        def out_wait(slot):
            pltpu.make_async_copy(
                outv.at[slot], out_hbm.at[pl.ds(0, ROWS_PER_BLOCK)], osems[slot]
            ).wait()

        gather_start(0, 0)
        gather_start(1, 1)

        @pl.loop(0, BLOCKS_PER_WORKER // NBUF)
        def _(p):
            for slot in range(NBUF):
                i = p * NBUF + slot
                gather_wait(slot)

                @pl.when(i >= NBUF)
                def _():
                    out_wait(slot)

                rbuf = rows.at[slot]
                obuf = outv.at[slot]

                @plsc.parallel_loop(0, ROWS_PER_BLOCK, unroll=2)
                def _(r):
                    base = r * GROUP
                    for c in range(D // LANES):
                        sl = pl.ds(c * LANES, LANES)
                        obuf[r, sl] = _tree_max(
                            [rbuf[base + j, sl] for j in range(GROUP)]
                        )

                pltpu.async_copy(
                    obuf,
                    out_hbm.at[pl.ds((base_blk + i) * ROWS_PER_BLOCK,
                                     ROWS_PER_BLOCK)],
                    osems[slot],
                )

                @pl.when(i + NBUF < BLOCKS_PER_WORKER)
                def _():
                    gather_start(slot, i + NBUF)

        out_wait(0)
        out_wait(1)

    return kern(x_padded, idx_flat)


def kernel(x, pool_inds):
    x_padded = jnp.concatenate([x, jnp.zeros((1, D), dtype=x.dtype)], axis=0)
    idx_flat = pool_inds.astype(jnp.int32).reshape(N_OUT * GROUP)
    idx_flat = jnp.concatenate(
        [idx_flat, jnp.zeros(N_BLOCKS_PAD * IDX_PER_BLOCK - N_OUT * GROUP,
                             dtype=jnp.int32)]
    )
    return _sc_max_pool(x_padded, idx_flat)[:N_OUT]


# speedup vs baseline: 4.2578x; 1.1775x over previous
import functools

import jax
import jax.numpy as jnp
from jax import lax
from jax.experimental import pallas as pl
from jax.experimental.pallas import tpu as pltpu
from jax.experimental.pallas import tpu_sc as plsc

N_ROWS_IN = 50000
D = 128
N_OUT = 25000
GROUP = 16

LANES = 16
ROWS_PER_BLOCK = 8
IDX_PER_BLOCK = ROWS_PER_BLOCK * GROUP
N_BLOCKS = N_OUT // ROWS_PER_BLOCK
CNT_C0 = 124
CNT_C1 = 72
N_BLOCKS_PAD = 16 * (CNT_C0 + CNT_C1)
NBUF = 2


def _tree_max(vals):
    while len(vals) > 1:
        vals = [jnp.maximum(a, b) for a, b in zip(vals[::2], vals[1::2])]
    return vals[0]


def _sc_max_pool(x_padded, idx_flat):
    mesh = plsc.VectorSubcoreMesh(core_axis_name="c", subcore_axis_name="s")

    @functools.partial(
        pl.kernel,
        out_type=jax.ShapeDtypeStruct((N_OUT, D), jnp.float32),
        mesh=mesh,
        scratch_types=[
            pltpu.VMEM((max(CNT_C0, CNT_C1) * IDX_PER_BLOCK,), jnp.int32),
            pltpu.VMEM((NBUF, IDX_PER_BLOCK, D), jnp.float32),
            pltpu.VMEM((NBUF, ROWS_PER_BLOCK, D), jnp.float32),
            pltpu.SemaphoreType.DMA,
            pltpu.SemaphoreType.DMA,
            pltpu.SemaphoreType.DMA,
            pltpu.SemaphoreType.DMA,
        ],
    )
    def kern(x_hbm, idx_hbm, out_hbm, idx_all, rows, outv, g0, g1, o0, o1):
        gsems = [g0, g1]
        osems = [o0, o1]
        c = lax.axis_index("c")
        s = lax.axis_index("s")

        def run(base_blk, nblk):
            pltpu.sync_copy(
                idx_hbm.at[pl.ds(base_blk * IDX_PER_BLOCK, nblk * IDX_PER_BLOCK)],
                idx_all.at[pl.ds(0, nblk * IDX_PER_BLOCK)],
            )

            def gather_start(slot, i):
                idx_slice = idx_all.at[pl.ds(i * IDX_PER_BLOCK, IDX_PER_BLOCK)]
                pltpu.async_copy(x_hbm.at[idx_slice], rows.at[slot], gsems[slot])

            def gather_wait(slot):
                pltpu.make_async_copy(
                    x_hbm.at[pl.ds(0, IDX_PER_BLOCK)], rows.at[slot], gsems[slot]
                ).wait()

            def out_wait(slot):
                pltpu.make_async_copy(
                    outv.at[slot], out_hbm.at[pl.ds(0, ROWS_PER_BLOCK)], osems[slot]
                ).wait()

            gather_start(0, 0)
            gather_start(1, 1)

            @pl.loop(0, nblk // NBUF)
            def _(p):
                for slot in range(NBUF):
                    i = p * NBUF + slot
                    b = base_blk + i
                    gather_wait(slot)

                    @pl.when(jnp.logical_and(i >= NBUF, b - NBUF < N_BLOCKS))
                    def _():
                        out_wait(slot)

                    rbuf = rows.at[slot]
                    obuf = outv.at[slot]

                    @plsc.parallel_loop(0, ROWS_PER_BLOCK, unroll=2)
                    def _(r):
                        base = r * GROUP
                        for chunk in range(D // LANES):
                            sl = pl.ds(chunk * LANES, LANES)
                            obuf[r, sl] = _tree_max(
                                [rbuf[base + j, sl] for j in range(GROUP)]
                            )

                    @pl.when(b < N_BLOCKS)
                    def _():
                        pltpu.async_copy(
                            obuf,
                            out_hbm.at[pl.ds(b * ROWS_PER_BLOCK, ROWS_PER_BLOCK)],
                            osems[slot],
                        )

                    @pl.when(i + NBUF < nblk)
                    def _():
                        gather_start(slot, i + NBUF)

            for i in (nblk - NBUF, nblk - 1):
                @pl.when(base_blk + i < N_BLOCKS)
                def _(i=i):
                    out_wait(i % NBUF)

        @pl.when(c == 0)
        def _():
            run(s * CNT_C0, CNT_C0)

        @pl.when(c == 1)
        def _():
            run(16 * CNT_C0 + s * CNT_C1, CNT_C1)

    return kern(x_padded, idx_flat)


def kernel(x, pool_inds):
    x_padded = jnp.concatenate([x, jnp.zeros((1, D), dtype=x.dtype)], axis=0)
    idx_flat = pool_inds.astype(jnp.int32).reshape(N_OUT * GROUP)
    idx_flat = jnp.concatenate(
        [idx_flat, jnp.zeros(N_BLOCKS_PAD * IDX_PER_BLOCK - N_OUT * GROUP,
                             dtype=jnp.int32)]
    )
    return _sc_max_pool(x_padded, idx_flat)
